# baseline (speedup 1.0000x reference)
import functools

import jax
import jax.numpy as jnp
from jax import lax
from jax.experimental import pallas as pl
from jax.experimental.pallas import tpu as pltpu
from jax.experimental.pallas import tpu_sc as plsc

BATCH = 16384
DIM = 64

NUM_CORES = 2
NUM_SUBCORES = 16
NUM_WORKERS = NUM_CORES * NUM_SUBCORES
BPW = BATCH // NUM_WORKERS
GROUP = 8
NGROUPS = BPW // GROUP
LANES = 16
CHUNK = 128
NCHUNK = BPW // CHUNK


def _gather_blocks(nids, table):
    mesh = plsc.VectorSubcoreMesh(core_axis_name="c", subcore_axis_name="s")

    @functools.partial(
        pl.kernel,
        mesh=mesh,
        out_type=jax.ShapeDtypeStruct((BATCH, DIM), jnp.float32),
        scratch_types=[
            pltpu.VMEM((BPW,), jnp.int32),
            pltpu.VMEM((2, GROUP, 8, DIM), jnp.float32),
            pltpu.VMEM((BPW, DIM), jnp.float32),
            pltpu.SemaphoreType.DMA,
            pltpu.SemaphoreType.DMA,
            pltpu.SemaphoreType.DMA,
        ],
    )
    def k(nids_hbm, t_hbm, out_hbm, idx_v, blk, rows_v, sem_i, sem_a, sem_b):
        wid = lax.axis_index("s") * NUM_CORES + lax.axis_index("c")
        base = wid * BPW
        pltpu.async_copy(nids_hbm.at[pl.ds(base, BPW)], idx_v, sem_i).wait()

        sems = (sem_a, sem_b)

        def fire(g, slot):
            ivec = idx_v[pl.ds(g * GROUP, GROUP)]
            for j in range(GROUP):
                s = ivec[j]
                b8 = pl.multiple_of((s >> 3) << 3, 8)
                pltpu.async_copy(t_hbm.at[pl.ds(b8, 8)], blk.at[slot, j], sems[slot])

        def drain_extract(g, slot):
            for j in range(GROUP):
                pltpu.make_async_copy(t_hbm.at[pl.ds(0, 8)], blk.at[slot, j], sems[slot]).wait()
            ivec = idx_v[pl.ds(g * GROUP, GROUP)]
            for j in range(GROUP):
                r = ivec[j] & 7
                row = g * GROUP + j
                for h in range(DIM // LANES):
                    rows_v[row, pl.ds(h * LANES, LANES)] = blk[slot, j, r, pl.ds(h * LANES, LANES)]

        fire(0, 0)

        def body(t, _):
            g0 = t * 2
            fire(g0 + 1, 1)
            drain_extract(g0, 0)

            @pl.when(g0 + 2 < NGROUPS)
            def _():
                fire(g0 + 2, 0)

            drain_extract(g0 + 1, 1)
            return ()

        lax.fori_loop(0, NGROUPS // 2, body, ())
        pltpu.sync_copy(rows_v, out_hbm.at[pl.ds(base, BPW)])

    return k(nids, table)


def _gather_stream(nids, table):
    mesh = plsc.VectorSubcoreMesh(core_axis_name="c", subcore_axis_name="s")

    @functools.partial(
        pl.kernel,
        mesh=mesh,
        out_type=jax.ShapeDtypeStruct((BATCH, DIM), jnp.float32),
        scratch_types=[
            pltpu.VMEM((NCHUNK, CHUNK), jnp.int32),
            pltpu.VMEM((BPW, DIM), jnp.float32),
            pltpu.SemaphoreType.DMA,
        ],
        compiler_params=pltpu.CompilerParams(use_tc_tiling_on_sc=False),
    )
    def k(nids_hbm, t_hbm, out_hbm, idx_v, rows_v, sem):
        wid = lax.axis_index("s") * NUM_CORES + lax.axis_index("c")
        base = wid * BPW
        for j in range(NCHUNK):
            pltpu.sync_copy(nids_hbm.at[pl.ds(base + j * CHUNK, CHUNK)], idx_v.at[j])
        cps = [
            pltpu.async_copy(
                t_hbm.at[idx_v.at[j]], rows_v.at[pl.ds(j * CHUNK, CHUNK)], sem
            )
            for j in range(NCHUNK)
        ]
        for cp in cps:
            cp.wait()
        pltpu.sync_copy(rows_v, out_hbm.at[pl.ds(base, BPW)])

    return k(nids, table)


def kernel(nids, is_start, directed, u_emb, v_emb):
    nids32 = nids.astype(jnp.int32)
    u_rows = _gather_stream(nids32, u_emb)
    v_rows = _gather_stream(nids32, v_emb)
    return jnp.concatenate((u_rows, v_rows), axis=1)


# speedup vs baseline: 1.5162x; 1.5162x over previous
import functools

import jax
import jax.numpy as jnp
from jax import lax
from jax.experimental import pallas as pl
from jax.experimental.pallas import tpu as pltpu
from jax.experimental.pallas import tpu_sc as plsc

BATCH = 16384
DIM = 64

NUM_CORES = 2
NUM_SUBCORES = 16
NUM_WORKERS = NUM_CORES * NUM_SUBCORES
BPW = BATCH // NUM_WORKERS
GROUP = 8
NGROUPS = BPW // GROUP
DEPTH = 4
LANES = 16


def _gather_cat(nids, u_emb, v_emb):
    mesh = plsc.VectorSubcoreMesh(core_axis_name="c", subcore_axis_name="s")

    @functools.partial(
        pl.kernel,
        mesh=mesh,
        out_type=jax.ShapeDtypeStruct((BATCH, 2 * DIM), jnp.float32),
        scratch_types=[
            pltpu.VMEM((BPW,), jnp.int32),
            pltpu.VMEM((DEPTH, GROUP, 8, DIM), jnp.float32),
            pltpu.VMEM((DEPTH, GROUP, 8, DIM), jnp.float32),
            pltpu.VMEM((DEPTH, GROUP, 2 * DIM), jnp.float32),
            pltpu.SemaphoreType.DMA,
            pltpu.SemaphoreType.DMA,
            pltpu.SemaphoreType.DMA,
            pltpu.SemaphoreType.DMA,
            pltpu.SemaphoreType.DMA,
            pltpu.SemaphoreType.DMA,
            pltpu.SemaphoreType.DMA,
            pltpu.SemaphoreType.DMA,
            pltpu.SemaphoreType.DMA,
        ],
    )
    def k(nids_hbm, u_hbm, v_hbm, out_hbm, idx_v, blk_u, blk_v, stage,
          sem_i, s0, s1, s2, s3, o0, o1, o2, o3):
        wid = lax.axis_index("s") * NUM_CORES + lax.axis_index("c")
        base = wid * BPW
        pltpu.async_copy(nids_hbm.at[pl.ds(base, BPW)], idx_v, sem_i).wait()

        sems = (s0, s1, s2, s3)
        osems = (o0, o1, o2, o3)

        def fire(g, slot):
            ivec = idx_v[pl.ds(g * GROUP, GROUP)]
            for j in range(GROUP):
                s = ivec[j]
                b8 = pl.multiple_of((s >> 3) << 3, 8)
                pltpu.async_copy(u_hbm.at[pl.ds(b8, 8)], blk_u.at[slot, j], sems[slot])
                pltpu.async_copy(v_hbm.at[pl.ds(b8, 8)], blk_v.at[slot, j], sems[slot])

        def drain_extract(g, slot):
            for j in range(GROUP):
                pltpu.make_async_copy(u_hbm.at[pl.ds(0, 8)], blk_u.at[slot, j], sems[slot]).wait()
                pltpu.make_async_copy(v_hbm.at[pl.ds(0, 8)], blk_v.at[slot, j], sems[slot]).wait()
            ivec = idx_v[pl.ds(g * GROUP, GROUP)]
            for j in range(GROUP):
                r = ivec[j] & 7
                for h in range(DIM // LANES):
                    stage[slot, j, pl.ds(h * LANES, LANES)] = blk_u[slot, j, r, pl.ds(h * LANES, LANES)]
                    stage[slot, j, pl.ds(DIM + h * LANES, LANES)] = blk_v[slot, j, r, pl.ds(h * LANES, LANES)]
            pltpu.async_copy(stage.at[slot], out_hbm.at[pl.ds(base + g * GROUP, GROUP)], osems[slot])

        for p in range(DEPTH - 1):
            fire(p, p)

        def body(t, _):
            for ph in range(DEPTH):
                g = t * DEPTH + ph

                @pl.when(t > 0)
                def _():
                    pltpu.make_async_copy(
                        stage.at[ph], out_hbm.at[pl.ds(0, GROUP)], osems[ph]
                    ).wait()

                @pl.when(g + DEPTH - 1 < NGROUPS)
                def _():
                    fire(g + DEPTH - 1, (ph + DEPTH - 1) % DEPTH)

                drain_extract(g, ph)
            return ()

        lax.fori_loop(0, NGROUPS // DEPTH, body, ())
        for p in range(DEPTH):
            pltpu.make_async_copy(stage.at[p], out_hbm.at[pl.ds(0, GROUP)], osems[p]).wait()

    return k(nids, u_emb, v_emb)


def kernel(nids, is_start, directed, u_emb, v_emb):
    return _gather_cat(nids.astype(jnp.int32), u_emb, v_emb)
